# baseline (speedup 1.0000x reference)
import functools

import jax
import jax.numpy as jnp
from jax.experimental import pallas as pl

_MAX_REL = 125
_VOCAB = 2 * _MAX_REL + 1
_BLK = 1024


def _body(x_ref, table_ref, out_ref, *, S):
    s = pl.program_id(0)
    blk = out_ref.shape[1]
    kdim = table_ref.shape[0]
    jj = s * blk + jax.lax.broadcasted_iota(jnp.int32, (blk, kdim), 0)
    kk = jax.lax.broadcasted_iota(jnp.int32, (blk, kdim), 1)
    km = kk - _MAX_REL
    hi = jnp.where(kk == _VOCAB - 1, S + _MAX_REL, km)
    lo = jnp.where(kk == 0, -(S + _MAX_REL), km)
    cnt = jnp.maximum(0, jnp.minimum(jj, hi) - jnp.maximum(jj - (S - 1), lo) + 1)
    c = cnt.astype(jnp.float32) * (1.0 / S)
    pooled = jax.lax.dot_general(
        c, table_ref[...],
        dimension_numbers=(((1,), (0,)), ((), ())),
        preferred_element_type=jnp.float32,
    )
    out_ref[...] = x_ref[...] + pooled[None, :, :]


def kernel(x, table):
    B, S, d = x.shape
    V = table.shape[0]
    grid = (S // _BLK,)
    body = functools.partial(_body, S=S)
    return pl.pallas_call(
        body,
        grid=grid,
        in_specs=[
            pl.BlockSpec((B, _BLK, d), lambda s: (0, s, 0)),
            pl.BlockSpec((V, d), lambda s: (0, 0)),
        ],
        out_specs=pl.BlockSpec((B, _BLK, d), lambda s: (0, s, 0)),
        out_shape=jax.ShapeDtypeStruct((B, S, d), x.dtype),
    )(x, table)
